# baseline (speedup 1.0000x reference)
import functools

import jax
import jax.numpy as jnp
from jax.experimental import pallas as pl
from jax.experimental.pallas import tpu as pltpu


def _ce_kl_body(logit_ref, cap_ref, len_ref, w_ref,
                mu_ref, s2_ref, mup_ref, s2p_ref,
                ce_out_ref, kl_out_ref,
                acc_ref, cnt_ref, *, nb, batch, t_len):
    step = pl.program_id(0)

    x = logit_ref[...]
    bb, tt, vv = x.shape

    m = jnp.max(x, axis=2)
    s = jnp.sum(jnp.exp(x - m[:, :, None]), axis=2)
    lse = m + jnp.log(s)

    tgt = cap_ref[:, 1:]
    iota_v = jax.lax.broadcasted_iota(jnp.int32, (bb, tt, vv), 2)
    picked = jnp.sum(jnp.where(iota_v == tgt[:, :, None], x, 0.0), axis=2)

    w = w_ref[:, 0]
    lengths = len_ref[:, 0] - 1
    iota_t = jax.lax.broadcasted_iota(jnp.int32, (bb, tt), 1)
    mask = (iota_t < lengths[:, None]).astype(jnp.float32)

    val = (picked - lse) * w[:, None]
    block_sum = jnp.sum(val * mask)
    block_cnt = jnp.sum(mask)

    @pl.when(step == 0)
    def _init():
        acc_ref[0] = 0.0
        cnt_ref[0] = 0.0
        mu = mu_ref[...]
        s2 = s2_ref[...]
        mup = mup_ref[...]
        s2p = s2p_ref[...]
        kl_terms = (1.0 + s2 - s2p - jnp.exp(s2 - s2p)
                    - (mu - mup) ** 2 * jnp.exp(-s2p))
        kl_out_ref[0, 0] = -0.5 * jnp.sum(kl_terms) / batch

    acc_ref[0] += block_sum
    cnt_ref[0] += block_cnt

    @pl.when(step == nb - 1)
    def _fin():
        ce_out_ref[0, 0] = -acc_ref[0] / cnt_ref[0]


def kernel(logit, mu, sigma2, mu_pri, sigma2_pri, cap, cap_len, weight):
    B, T, V = logit.shape
    D = mu.shape[1]
    BB = 8
    NB = B // BB

    cap_i = cap.astype(jnp.int32)
    len_i = cap_len.astype(jnp.int32).reshape(B, 1)
    w_2d = weight.reshape(B, 1)

    body = functools.partial(_ce_kl_body, nb=NB, batch=B, t_len=T)

    ce, kl = pl.pallas_call(
        body,
        grid=(NB,),
        in_specs=[
            pl.BlockSpec((BB, T, V), lambda i: (i, 0, 0)),
            pl.BlockSpec((BB, T + 1), lambda i: (i, 0)),
            pl.BlockSpec((BB, 1), lambda i: (i, 0)),
            pl.BlockSpec((BB, 1), lambda i: (i, 0)),
            pl.BlockSpec((B, D), lambda i: (0, 0)),
            pl.BlockSpec((B, D), lambda i: (0, 0)),
            pl.BlockSpec((B, D), lambda i: (0, 0)),
            pl.BlockSpec((B, D), lambda i: (0, 0)),
        ],
        out_specs=[
            pl.BlockSpec((1, 1), lambda i: (0, 0), memory_space=pltpu.SMEM),
            pl.BlockSpec((1, 1), lambda i: (0, 0), memory_space=pltpu.SMEM),
        ],
        out_shape=[
            jax.ShapeDtypeStruct((1, 1), jnp.float32),
            jax.ShapeDtypeStruct((1, 1), jnp.float32),
        ],
        scratch_shapes=[
            pltpu.SMEM((1,), jnp.float32),
            pltpu.SMEM((1,), jnp.float32),
        ],
    )(logit, cap_i, len_i, w_2d, mu, sigma2, mu_pri, sigma2_pri)

    return (ce.reshape(()), kl.reshape(()))
